# baseline (speedup 1.0000x reference)
import functools

import jax
import jax.numpy as jnp
import numpy as np
from jax.experimental import pallas as pl
from jax.experimental.pallas import tpu as pltpu
from jax.experimental.shard_map import shard_map
from jax.sharding import Mesh, PartitionSpec as P

B, D, O, E = 16384, 768, 768, 8
BLOCK_B = 1024


def _body(x_ref, ws_ref, bs_ref, w_ref, bp_ref, o_ref, wb_ref):
    @pl.when(pl.program_id(0) == 0)
    def _cast_weights():
        wb_ref[...] = w_ref[...].astype(jnp.bfloat16)

    x = x_ref[...].astype(jnp.bfloat16)
    logits = jnp.dot(
        x, ws_ref[...].astype(jnp.bfloat16), preferred_element_type=jnp.float32
    )
    logits = logits + bs_ref[...]
    p = jax.nn.softmax(logits, axis=-1)
    acc = jnp.zeros((x.shape[0], O), jnp.float32)
    for e in range(E):
        y_e = jnp.dot(x, wb_ref[e], preferred_element_type=jnp.float32)
        acc += p[:, e : e + 1] * (y_e + bp_ref[e : e + 1, :])
    o_ref[...] = acc


def _drn_block(x, ws, bs, wp, bp):
    b_local = x.shape[0]
    grid = (b_local // BLOCK_B,)
    return pl.pallas_call(
        _body,
        grid=grid,
        in_specs=[
            pl.BlockSpec((BLOCK_B, D), lambda i: (i, 0)),
            pl.BlockSpec((D, E), lambda i: (0, 0)),
            pl.BlockSpec((1, E), lambda i: (0, 0)),
            pl.BlockSpec((E, D, O), lambda i: (0, 0, 0)),
            pl.BlockSpec((E, O), lambda i: (0, 0)),
        ],
        out_specs=pl.BlockSpec((BLOCK_B, O), lambda i: (i, 0)),
        out_shape=jax.ShapeDtypeStruct((b_local, O), jnp.float32),
        scratch_shapes=[pltpu.VMEM((E, D, O), jnp.bfloat16)],
        compiler_params=pltpu.CompilerParams(
            dimension_semantics=("arbitrary",),
        ),
    )(x, ws, bs, wp, bp)


@functools.lru_cache(maxsize=1)
def _sharded_fn():
    devs = jax.devices()
    n = 2 if len(devs) >= 2 and B % (2 * BLOCK_B) == 0 else 1
    mesh = Mesh(np.array(devs[:n]), ("dp",))
    return shard_map(
        _drn_block,
        mesh=mesh,
        in_specs=(P("dp", None), P(None, None), P(None, None), P(None, None, None), P(None, None)),
        out_specs=P("dp", None),
        check_rep=False,
    )


def kernel(x, W_sel, b_sel, W_pops, b_pops):
    return _sharded_fn()(x, W_sel, b_sel.reshape(1, E), W_pops, b_pops)


# speedup vs baseline: 2.9482x; 2.9482x over previous
import jax
import jax.numpy as jnp
from jax.experimental import pallas as pl
from jax.experimental.pallas import tpu as pltpu

B, D, O, E = 16384, 768, 768, 8
BLOCK_B = 1024


def _body(x_ref, ws_ref, bs_ref, w_ref, bp_ref, o_ref, wb_ref):
    @pl.when(pl.program_id(0) == 0)
    def _cast_weights():
        wb_ref[...] = w_ref[...].astype(jnp.bfloat16)

    x = x_ref[...].astype(jnp.bfloat16)
    logits = jnp.dot(
        x, ws_ref[...].astype(jnp.bfloat16), preferred_element_type=jnp.float32
    )
    logits = logits + bs_ref[...]
    p = jax.nn.softmax(logits, axis=-1)
    acc = jnp.zeros((x.shape[0], O), jnp.float32)
    for e in range(E):
        y_e = jnp.dot(x, wb_ref[e], preferred_element_type=jnp.float32)
        acc += p[:, e : e + 1] * (y_e + bp_ref[e : e + 1, :])
    o_ref[...] = acc


def kernel(x, W_sel, b_sel, W_pops, b_pops):
    grid = (B // BLOCK_B,)
    return pl.pallas_call(
        _body,
        grid=grid,
        in_specs=[
            pl.BlockSpec((BLOCK_B, D), lambda i: (i, 0)),
            pl.BlockSpec((D, E), lambda i: (0, 0)),
            pl.BlockSpec((1, E), lambda i: (0, 0)),
            pl.BlockSpec((E, D, O), lambda i: (0, 0, 0)),
            pl.BlockSpec((E, O), lambda i: (0, 0)),
        ],
        out_specs=pl.BlockSpec((BLOCK_B, O), lambda i: (i, 0)),
        out_shape=jax.ShapeDtypeStruct((B, O), jnp.float32),
        scratch_shapes=[pltpu.VMEM((E, D, O), jnp.bfloat16)],
        compiler_params=pltpu.CompilerParams(
            dimension_semantics=("arbitrary",),
        ),
    )(x, W_sel, b_sel.reshape(1, E), W_pops, b_pops)
